# baseline (speedup 1.0000x reference)
import jax
import jax.numpy as jnp
from jax import lax
from jax.experimental import pallas as pl
from jax.experimental.pallas import tpu as pltpu

B = 4096
S = 200
D = 64
NSPAN = S // 2


def _body(x_ref, s_ref, e_ref, l_ref, sem):
    l_ref[...] = jnp.full((B, NSPAN), 2, jnp.int32)
    x4 = x_ref.reshape(B, NSPAN, 2, D)
    pltpu.make_async_copy(x4.at[:, :, 0, :], s_ref, sem).start()
    pltpu.make_async_copy(x4.at[:, :, 1, :], e_ref, sem).start()
    pltpu.make_async_copy(x4.at[:, :, 0, :], s_ref, sem).wait()
    pltpu.make_async_copy(x4.at[:, :, 1, :], e_ref, sem).wait()


@jax.jit
def kernel(input):
    return pl.pallas_call(
        _body,
        in_specs=[pl.BlockSpec(memory_space=pl.ANY)],
        out_specs=(
            pl.BlockSpec(memory_space=pl.ANY),
            pl.BlockSpec(memory_space=pl.ANY),
            pl.BlockSpec((B, NSPAN), lambda: (0, 0)),
        ),
        out_shape=(
            jax.ShapeDtypeStruct((B, NSPAN, D), jnp.float32),
            jax.ShapeDtypeStruct((B, NSPAN, D), jnp.float32),
            jax.ShapeDtypeStruct((B, NSPAN), jnp.int32),
        ),
        scratch_shapes=[pltpu.SemaphoreType.DMA],
    )(input)


# speedup vs baseline: 14.8738x; 14.8738x over previous
import jax
import jax.numpy as jnp
from jax import lax
from jax.experimental import pallas as pl
from jax.experimental.pallas import tpu as pltpu

B = 4096
S = 200
D = 64
NSPAN = S // 2
BT = 16


def _body(x_ref, s_ref, e_ref, l_ref):
    s_ref[...] = x_ref[:, pl.dslice(0, NSPAN, 2), :]
    e_ref[...] = x_ref[:, pl.dslice(1, NSPAN, 2), :]
    l_ref[...] = jnp.full((BT, NSPAN), 2, jnp.int32)


@jax.jit
def kernel(input):
    return pl.pallas_call(
        _body,
        grid=(B // BT,),
        in_specs=[pl.BlockSpec((BT, S, D), lambda i: (i, 0, 0))],
        out_specs=(
            pl.BlockSpec((BT, NSPAN, D), lambda i: (i, 0, 0)),
            pl.BlockSpec((BT, NSPAN, D), lambda i: (i, 0, 0)),
            pl.BlockSpec((BT, NSPAN), lambda i: (i, 0)),
        ),
        out_shape=(
            jax.ShapeDtypeStruct((B, NSPAN, D), jnp.float32),
            jax.ShapeDtypeStruct((B, NSPAN, D), jnp.float32),
            jax.ShapeDtypeStruct((B, NSPAN), jnp.int32),
        ),
    )(input)


# speedup vs baseline: 15.6395x; 1.0515x over previous
import jax
import jax.numpy as jnp
from jax import lax
from jax.experimental import pallas as pl
from jax.experimental.pallas import tpu as pltpu

B = 4096
S = 200
D = 64
NSPAN = S // 2
BT = 128


def _body(x_ref, s_ref, e_ref, l_ref):
    s_ref[...] = x_ref[:, pl.dslice(0, NSPAN, 2), :]
    e_ref[...] = x_ref[:, pl.dslice(1, NSPAN, 2), :]
    l_ref[...] = jnp.full((BT, NSPAN), 2, jnp.int32)


@jax.jit
def kernel(input):
    return pl.pallas_call(
        _body,
        grid=(B // BT,),
        in_specs=[pl.BlockSpec((BT, S, D), lambda i: (i, 0, 0))],
        out_specs=(
            pl.BlockSpec((BT, NSPAN, D), lambda i: (i, 0, 0)),
            pl.BlockSpec((BT, NSPAN, D), lambda i: (i, 0, 0)),
            pl.BlockSpec((BT, NSPAN), lambda i: (i, 0)),
        ),
        out_shape=(
            jax.ShapeDtypeStruct((B, NSPAN, D), jnp.float32),
            jax.ShapeDtypeStruct((B, NSPAN, D), jnp.float32),
            jax.ShapeDtypeStruct((B, NSPAN), jnp.int32),
        ),
    )(input)
